# baseline (speedup 1.0000x reference)
import functools

import jax
import jax.numpy as jnp
from jax import lax
from jax.experimental import pallas as pl
from jax.experimental.pallas import tpu as pltpu
from jax.experimental.pallas import tpu_sc as plsc

N_NODES = 10000
N_EDGES = 320000
D_IN = 128
D_HID = 16
D_OUT = 64

NC = 2
NS = 16
NW = NC * NS
CHUNK = 128
CPW = -(-N_EDGES // (CHUNK * NW))
N_CHUNKS = CPW * NW
E_PAD = N_CHUNKS * CHUNK
N_PAD = 10240
RPS = N_PAD // NS

_HIGH = jax.lax.Precision.HIGHEST
_MESH = dict(core_axis_name="c", subcore_axis_name="s")
_DOT = (((1,), (0,)), ((), ()))
_SC_PARAMS = pltpu.CompilerParams(use_tc_tiling_on_sc=False)


def _sc_degree(dst2d):
    @functools.partial(
        pl.kernel,
        out_type=jax.ShapeDtypeStruct((NC, N_PAD, D_HID), jnp.float32),
        mesh=plsc.VectorSubcoreMesh(**_MESH),
        scratch_types=[
            pltpu.VMEM((CPW, CHUNK), jnp.int32),
            pltpu.VMEM((CHUNK, D_HID), jnp.float32),
            pltpu.VMEM_SHARED((N_PAD, D_HID), jnp.float32),
        ],
        compiler_params=_SC_PARAMS,
    )
    def deg_kernel(dst_hbm, out_hbm, di_v, ones_v, acc_sh):
        c = lax.axis_index("c")
        s = lax.axis_index("s")
        w = c * NS + s
        row0 = s * RPS
        pltpu.sync_copy(dst_hbm.at[pl.ds(w * CPW, CPW)], di_v)

        @pl.loop(0, CHUNK)
        def _(i):
            ones_v[i] = jnp.zeros((D_HID,), jnp.float32)

        for k in range(RPS // CHUNK):
            pltpu.sync_copy(ones_v, acc_sh.at[pl.ds(row0 + k * CHUNK, CHUNK)])

        @pl.loop(0, CHUNK)
        def _(i):
            ones_v[i] = jnp.ones((D_HID,), jnp.float32)

        plsc.subcore_barrier()

        @pl.loop(0, CPW)
        def _(i):
            pltpu.sync_copy(ones_v, acc_sh.at[di_v.at[i]], add=True)

        plsc.subcore_barrier()
        pltpu.sync_copy(acc_sh.at[pl.ds(row0, RPS)],
                        out_hbm.at[c, pl.ds(row0, RPS)])

    return deg_kernel(dst2d)


def _sc_scatter(table, src2d, dst2d, d):
    @functools.partial(
        pl.kernel,
        out_type=jax.ShapeDtypeStruct((NC, N_PAD, d), jnp.bfloat16),
        mesh=plsc.VectorSubcoreMesh(**_MESH),
        scratch_types=[
            pltpu.VMEM((CPW, CHUNK), jnp.int32),
            pltpu.VMEM((CPW, CHUNK), jnp.int32),
            pltpu.VMEM((CHUNK, d), jnp.bfloat16),
            pltpu.VMEM((CHUNK, d), jnp.bfloat16),
            pltpu.VMEM_SHARED((N_PAD, d), jnp.bfloat16),
            pltpu.SemaphoreType.DMA,
            pltpu.SemaphoreType.DMA,
        ],
        compiler_params=_SC_PARAMS,
    )
    def scat_kernel(tab_hbm, src_hbm, dst_hbm, out_hbm,
                    si_v, di_v, buf_a, buf_b, acc_sh, sem_a, sem_b):
        c = lax.axis_index("c")
        s = lax.axis_index("s")
        w = c * NS + s
        row0 = s * RPS
        pltpu.sync_copy(src_hbm.at[pl.ds(w * CPW, CPW)], si_v)
        pltpu.sync_copy(dst_hbm.at[pl.ds(w * CPW, CPW)], di_v)

        @pl.loop(0, CHUNK // 2)
        def _(i):
            for col in range(d // D_HID):
                buf_a[pl.ds(2 * i, 2), pl.ds(col * D_HID, D_HID)] = (
                    jnp.zeros((2, D_HID), jnp.bfloat16))

        for k in range(RPS // CHUNK):
            pltpu.sync_copy(buf_a, acc_sh.at[pl.ds(row0 + k * CHUNK, CHUNK)])
        plsc.subcore_barrier()

        def gather(i, buf, sem):
            pltpu.async_copy(tab_hbm.at[si_v.at[i]], buf, sem)

        def drain_scatter(i, buf, sem):
            pltpu.make_async_copy(tab_hbm.at[si_v.at[i]], buf, sem).wait()
            pltpu.sync_copy(buf, acc_sh.at[di_v.at[i]], add=True)

        gather(0, buf_a, sem_a)

        @pl.loop(0, CPW - 1, step=2)
        def _(i):
            gather(i + 1, buf_b, sem_b)
            drain_scatter(i, buf_a, sem_a)
            gather(i + 2, buf_a, sem_a)
            drain_scatter(i + 1, buf_b, sem_b)

        drain_scatter(CPW - 1, buf_a, sem_a)

        plsc.subcore_barrier()
        pltpu.sync_copy(acc_sh.at[pl.ds(row0, RPS)],
                        out_hbm.at[c, pl.ds(row0, RPS)])

    return scat_kernel(table, src2d, dst2d)


_BR = 1000


def _tc_first(x, w1, deg_p):
    def body(x_ref, w_ref, dp_ref, u_ref, hn_ref):
        h1 = lax.dot_general(x_ref[...], w_ref[...], _DOT, precision=_HIGH,
                             preferred_element_type=jnp.float32)
        u = lax.rsqrt(dp_ref[0] + dp_ref[1] + 1.0)
        u_ref[...] = u
        hn_ref[...] = (u * h1).astype(jnp.bfloat16)

    return pl.pallas_call(
        body,
        grid=(N_NODES // _BR,),
        in_specs=[pl.BlockSpec((_BR, D_IN), lambda i: (i, 0)),
                  pl.BlockSpec((D_IN, D_HID), lambda i: (0, 0)),
                  pl.BlockSpec((NC, _BR, D_HID), lambda i: (0, i, 0))],
        out_specs=[pl.BlockSpec((_BR, D_HID), lambda i: (i, 0)),
                   pl.BlockSpec((_BR, D_HID), lambda i: (i, 0))],
        out_shape=[jax.ShapeDtypeStruct((N_NODES, D_HID), jnp.float32),
                   jax.ShapeDtypeStruct((N_NODES, D_HID), jnp.bfloat16)],
    )(x, w1, deg_p)


def _tc_mid(p1, hn1, u16, b1, w2):
    def body(p_ref, hn_ref, u_ref, b_ref, w_ref, hn2_ref, u64_ref):
        s1 = (p_ref[0] + p_ref[1]).astype(jnp.float32) + hn_ref[...].astype(jnp.float32)
        pre = u_ref[...] * s1 + b_ref[...]
        o1 = jnp.maximum(pre, 0.0)
        h2 = lax.dot_general(o1, w_ref[...], _DOT, precision=_HIGH,
                             preferred_element_type=jnp.float32)
        sel = (lax.broadcasted_iota(jnp.int32, (D_HID, D_OUT), 0) == 0)
        u64 = lax.dot_general(u_ref[...], sel.astype(jnp.float32), _DOT,
                              precision=_HIGH, preferred_element_type=jnp.float32)
        u64_ref[...] = u64
        hn2_ref[...] = (u64 * h2).astype(jnp.bfloat16)

    return pl.pallas_call(
        body,
        grid=(N_NODES // _BR,),
        in_specs=[pl.BlockSpec((NC, _BR, D_HID), lambda i: (0, i, 0)),
                  pl.BlockSpec((_BR, D_HID), lambda i: (i, 0)),
                  pl.BlockSpec((_BR, D_HID), lambda i: (i, 0)),
                  pl.BlockSpec((1, D_HID), lambda i: (0, 0)),
                  pl.BlockSpec((D_HID, D_OUT), lambda i: (0, 0))],
        out_specs=[pl.BlockSpec((_BR, D_OUT), lambda i: (i, 0)),
                   pl.BlockSpec((_BR, D_OUT), lambda i: (i, 0))],
        out_shape=[jax.ShapeDtypeStruct((N_NODES, D_OUT), jnp.bfloat16),
                   jax.ShapeDtypeStruct((N_NODES, D_OUT), jnp.float32)],
    )(p1, hn1, u16, b1, w2)


def _tc_final(p2, hn2, u64, b2):
    def body(p_ref, hn_ref, u_ref, b_ref, o_ref):
        s2 = (p_ref[0] + p_ref[1]).astype(jnp.float32) + hn_ref[...].astype(jnp.float32)
        o_ref[...] = u_ref[...] * s2 + b_ref[...]

    return pl.pallas_call(
        body,
        grid=(N_NODES // _BR,),
        in_specs=[pl.BlockSpec((NC, _BR, D_OUT), lambda i: (0, i, 0)),
                  pl.BlockSpec((_BR, D_OUT), lambda i: (i, 0)),
                  pl.BlockSpec((_BR, D_OUT), lambda i: (i, 0)),
                  pl.BlockSpec((1, D_OUT), lambda i: (0, 0))],
        out_specs=pl.BlockSpec((_BR, D_OUT), lambda i: (i, 0)),
        out_shape=jax.ShapeDtypeStruct((N_NODES, D_OUT), jnp.float32),
    )(p2, hn2, u64, b2)


def kernel(x, edge_index, W1, b1, W2, b2):
    src = edge_index[0].astype(jnp.int32)
    dst = edge_index[1].astype(jnp.int32)
    pad = E_PAD - N_EDGES
    src2d = jnp.concatenate([src, jnp.zeros((pad,), jnp.int32)]).reshape(N_CHUNKS, CHUNK)
    dst2d = jnp.concatenate([dst, jnp.full((pad,), N_NODES, jnp.int32)]).reshape(N_CHUNKS, CHUNK)
    deg_p = _sc_degree(dst2d)
    u16, hn1 = _tc_first(x, W1, deg_p)
    p1 = _sc_scatter(hn1, src2d, dst2d, D_HID)
    hn2, u64 = _tc_mid(p1, hn1, u16, b1.reshape(1, D_HID), W2)
    p2 = _sc_scatter(hn2, src2d, dst2d, D_OUT)
    return _tc_final(p2, hn2, u64, b2.reshape(1, D_OUT))


# speedup vs baseline: 1.1286x; 1.1286x over previous
import functools

import jax
import jax.numpy as jnp
from jax import lax
from jax.experimental import pallas as pl
from jax.experimental.pallas import tpu as pltpu
from jax.experimental.pallas import tpu_sc as plsc

N_NODES = 10000
N_EDGES = 320000
D_IN = 128
D_HID = 16
D_OUT = 64

NC = 2
NS = 16
NW = NC * NS
CHUNK = 128
CPW = -(-N_EDGES // (CHUNK * NW))
N_CHUNKS = CPW * NW
E_PAD = N_CHUNKS * CHUNK
N_PAD = 10240
RPS = N_PAD // NS

_HIGH = jax.lax.Precision.HIGHEST
_MESH = dict(core_axis_name="c", subcore_axis_name="s")
_DOT = (((1,), (0,)), ((), ()))
_SC_PARAMS = pltpu.CompilerParams(use_tc_tiling_on_sc=False)


def _sc_degree(dst2d):
    @functools.partial(
        pl.kernel,
        out_type=jax.ShapeDtypeStruct((NC, N_PAD, D_HID), jnp.float32),
        mesh=plsc.VectorSubcoreMesh(**_MESH),
        scratch_types=[
            pltpu.VMEM((CPW, CHUNK), jnp.int32),
            pltpu.VMEM((CHUNK, D_HID), jnp.float32),
            pltpu.VMEM_SHARED((N_PAD, D_HID), jnp.float32),
        ],
        compiler_params=_SC_PARAMS,
    )
    def deg_kernel(dst_hbm, out_hbm, di_v, ones_v, acc_sh):
        c = lax.axis_index("c")
        s = lax.axis_index("s")
        w = c * NS + s
        row0 = s * RPS
        pltpu.sync_copy(dst_hbm.at[pl.ds(w * CPW, CPW)], di_v)

        @pl.loop(0, CHUNK)
        def _(i):
            ones_v[i] = jnp.zeros((D_HID,), jnp.float32)

        for k in range(RPS // CHUNK):
            pltpu.sync_copy(ones_v, acc_sh.at[pl.ds(row0 + k * CHUNK, CHUNK)])

        @pl.loop(0, CHUNK)
        def _(i):
            ones_v[i] = jnp.ones((D_HID,), jnp.float32)

        plsc.subcore_barrier()

        @pl.loop(0, CPW)
        def _(i):
            pltpu.sync_copy(ones_v, acc_sh.at[di_v.at[i]], add=True)

        plsc.subcore_barrier()
        pltpu.sync_copy(acc_sh.at[pl.ds(row0, RPS)],
                        out_hbm.at[c, pl.ds(row0, RPS)])

    return deg_kernel(dst2d)


def _sc_scatter(table, src2d, dst2d, d):
    @functools.partial(
        pl.kernel,
        out_type=jax.ShapeDtypeStruct((NC, N_PAD, d), jnp.bfloat16),
        mesh=plsc.VectorSubcoreMesh(**_MESH),
        scratch_types=[
            pltpu.VMEM((CPW, CHUNK), jnp.int32),
            pltpu.VMEM((CPW, CHUNK), jnp.int32),
            pltpu.VMEM((CHUNK, d), jnp.bfloat16),
            pltpu.VMEM((CHUNK, d), jnp.bfloat16),
            pltpu.VMEM_SHARED((N_PAD, d), jnp.bfloat16),
            pltpu.SemaphoreType.DMA,
            pltpu.SemaphoreType.DMA,
        ],
        compiler_params=_SC_PARAMS,
    )
    def scat_kernel(tab_hbm, src_hbm, dst_hbm, out_hbm,
                    si_v, di_v, buf_a, buf_b, acc_sh, sem_a, sem_b):
        c = lax.axis_index("c")
        s = lax.axis_index("s")
        w = c * NS + s
        row0 = s * RPS
        pltpu.sync_copy(src_hbm.at[pl.ds(w * CPW, CPW)], si_v)
        pltpu.sync_copy(dst_hbm.at[pl.ds(w * CPW, CPW)], di_v)

        @pl.loop(0, CHUNK // 2)
        def _(i):
            for col in range(d // D_HID):
                buf_a[pl.ds(2 * i, 2), pl.ds(col * D_HID, D_HID)] = (
                    jnp.zeros((2, D_HID), jnp.bfloat16))

        for k in range(RPS // CHUNK):
            pltpu.sync_copy(buf_a, acc_sh.at[pl.ds(row0 + k * CHUNK, CHUNK)])
        plsc.subcore_barrier()

        def gather(i, buf, sem):
            pltpu.async_copy(tab_hbm.at[si_v.at[i]], buf, sem)

        def drain_scatter(i, buf, sem):
            pltpu.make_async_copy(tab_hbm.at[si_v.at[i]], buf, sem).wait()
            pltpu.sync_copy(buf, acc_sh.at[di_v.at[i]], add=True)

        gather(0, buf_a, sem_a)

        @pl.loop(0, CPW - 1, step=2)
        def _(i):
            gather(i + 1, buf_b, sem_b)
            drain_scatter(i, buf_a, sem_a)
            gather(i + 2, buf_a, sem_a)
            drain_scatter(i + 1, buf_b, sem_b)

        drain_scatter(CPW - 1, buf_a, sem_a)

        plsc.subcore_barrier()
        pltpu.sync_copy(acc_sh.at[pl.ds(row0, RPS)],
                        out_hbm.at[c, pl.ds(row0, RPS)])

    return scat_kernel(table, src2d, dst2d)


_NB = 1024
_V16 = N_PAD * D_HID // 128
_V64 = N_PAD * D_OUT // 128
_B16 = _NB * D_HID // 128
_B64 = _NB * D_OUT // 128


def _tc_first(x_v, w1k, deg_v):
    def body(x_ref, w_ref, dp_ref, u_ref, hn_ref):
        h1v = lax.dot_general(x_ref[...], w_ref[...], _DOT, precision=_HIGH,
                              preferred_element_type=jnp.float32)
        u = lax.rsqrt(dp_ref[0] + dp_ref[1] + 1.0)
        u_ref[...] = u
        hn_ref[...] = (u * h1v).astype(jnp.bfloat16)

    return pl.pallas_call(
        body,
        grid=(N_PAD // _NB,),
        in_specs=[pl.BlockSpec((_B16, 8 * D_IN), lambda i: (i, 0)),
                  pl.BlockSpec((8 * D_IN, 128), lambda i: (0, 0)),
                  pl.BlockSpec((NC, _B16, 128), lambda i: (0, i, 0))],
        out_specs=[pl.BlockSpec((_B16, 128), lambda i: (i, 0)),
                   pl.BlockSpec((_B16, 128), lambda i: (i, 0))],
        out_shape=[jax.ShapeDtypeStruct((_V16, 128), jnp.float32),
                   jax.ShapeDtypeStruct((_V16, 128), jnp.bfloat16)],
    )(x_v, w1k, deg_v)


def _tc_mid(p1v, hn1v, u16v, b1t, w2k, selk):
    def body(p_ref, hn_ref, u_ref, b_ref, w_ref, sel_ref, hn2_ref, u64_ref):
        s1 = (p_ref[0] + p_ref[1]).astype(jnp.float32) + hn_ref[...].astype(jnp.float32)
        pre = u_ref[...] * s1 + b_ref[...]
        o1v = jnp.maximum(pre, 0.0)
        h2w = lax.dot_general(o1v, w_ref[...], _DOT, precision=_HIGH,
                              preferred_element_type=jnp.float32)
        u64w = lax.dot_general(u_ref[...], sel_ref[...], _DOT, precision=_HIGH,
                               preferred_element_type=jnp.float32)
        u64_ref[...] = jnp.reshape(u64w, (_B64, 128))
        hn2_ref[...] = jnp.reshape((u64w * h2w).astype(jnp.bfloat16), (_B64, 128))

    return pl.pallas_call(
        body,
        grid=(N_PAD // _NB,),
        in_specs=[pl.BlockSpec((NC, _B16, 128), lambda i: (0, i, 0)),
                  pl.BlockSpec((_B16, 128), lambda i: (i, 0)),
                  pl.BlockSpec((_B16, 128), lambda i: (i, 0)),
                  pl.BlockSpec((1, 128), lambda i: (0, 0)),
                  pl.BlockSpec((128, 8 * D_OUT), lambda i: (0, 0)),
                  pl.BlockSpec((128, 8 * D_OUT), lambda i: (0, 0))],
        out_specs=[pl.BlockSpec((_B64, 128), lambda i: (i, 0)),
                   pl.BlockSpec((_B64, 128), lambda i: (i, 0))],
        out_shape=[jax.ShapeDtypeStruct((_V64, 128), jnp.bfloat16),
                   jax.ShapeDtypeStruct((_V64, 128), jnp.float32)],
    )(p1v, hn1v, u16v, b1t, w2k, selk)


_FV = N_NODES * D_OUT // 128
_FB = 1000


def _tc_final(p2v, hn2v, u64v, b2t):
    def body(p_ref, hn_ref, u_ref, b_ref, o_ref):
        s2 = (p_ref[0] + p_ref[1]).astype(jnp.float32) + hn_ref[...].astype(jnp.float32)
        o_ref[...] = u_ref[...] * s2 + b_ref[...]

    return pl.pallas_call(
        body,
        grid=(_FV // _FB,),
        in_specs=[pl.BlockSpec((NC, _FB, 128), lambda i: (0, i, 0)),
                  pl.BlockSpec((_FB, 128), lambda i: (i, 0)),
                  pl.BlockSpec((_FB, 128), lambda i: (i, 0)),
                  pl.BlockSpec((1, 128), lambda i: (0, 0))],
        out_specs=pl.BlockSpec((_FB, 128), lambda i: (i, 0)),
        out_shape=jax.ShapeDtypeStruct((_FV, 128), jnp.float32),
    )(p2v, hn2v, u64v, b2t)


def kernel(x, edge_index, W1, b1, W2, b2):
    src = edge_index[0].astype(jnp.int32)
    dst = edge_index[1].astype(jnp.int32)
    pad = E_PAD - N_EDGES
    src2d = jnp.concatenate([src, jnp.zeros((pad,), jnp.int32)]).reshape(N_CHUNKS, CHUNK)
    dst2d = jnp.concatenate([dst, jnp.full((pad,), N_NODES, jnp.int32)]).reshape(N_CHUNKS, CHUNK)
    x_v = jnp.concatenate(
        [x, jnp.zeros((N_PAD - N_NODES, D_IN), jnp.float32)]).reshape(_V16, 8 * D_IN)
    eye8 = jnp.eye(8, dtype=jnp.float32)
    w1k = jnp.kron(eye8, W1)
    w2k = jnp.kron(eye8, W2)
    selk = jnp.kron(eye8, jnp.zeros((D_HID, D_OUT), jnp.float32).at[0].set(1.0))
    b1t = jnp.tile(b1, 128 // D_HID).reshape(1, 128)
    b2t = jnp.tile(b2, 128 // D_OUT).reshape(1, 128)

    deg_p = _sc_degree(dst2d)
    deg_v = jnp.reshape(deg_p, (NC, _V16, 128))
    u16v, hn1v = _tc_first(x_v, w1k, deg_v)
    p1 = _sc_scatter(hn1v.reshape(N_PAD, D_HID), src2d, dst2d, D_HID)
    p1v = jnp.reshape(p1, (NC, _V16, 128))
    hn2v, u64v = _tc_mid(p1v, hn1v, u16v, b1t, w2k, selk)
    p2 = _sc_scatter(hn2v.reshape(N_PAD, D_OUT), src2d, dst2d, D_OUT)
    p2v = jnp.reshape(p2, (NC, _V64, 128))
    out_v = _tc_final(p2v, hn2v, u64v, b2t)
    return jnp.reshape(out_v, (N_NODES, D_OUT))


# speedup vs baseline: 1.2234x; 1.0840x over previous
import functools

import jax
import jax.numpy as jnp
from jax import lax
from jax.experimental import pallas as pl
from jax.experimental.pallas import tpu as pltpu
from jax.experimental.pallas import tpu_sc as plsc

N_NODES = 10000
N_EDGES = 320000
D_IN = 128
D_HID = 16
D_OUT = 64

NC = 2
NS = 16
NW = NC * NS
CHUNK = 128
CPW = -(-N_EDGES // (CHUNK * NW))
CPW0_L1, CPW1_L1 = 82, 76
CPW0_L2, CPW1_L2 = 100, 58
_CPW0_MAX = max(CPW0_L1, CPW0_L2)
N_CHUNKS = CPW * NW
N_CHUNKROWS = NS * 2 * CPW + (_CPW0_MAX - min(CPW1_L1, CPW1_L2))
E_PAD = N_CHUNKROWS * CHUNK
N_PAD = 10240
RPS = N_PAD // NS

_HIGH = jax.lax.Precision.HIGHEST
_MESH = dict(core_axis_name="c", subcore_axis_name="s")
_DOT = (((1,), (0,)), ((), ()))
_SC_PARAMS = pltpu.CompilerParams(use_tc_tiling_on_sc=False)


def _sc_degree(dst2d):
    @functools.partial(
        pl.kernel,
        out_type=jax.ShapeDtypeStruct((NC, N_PAD, D_HID), jnp.float32),
        mesh=plsc.VectorSubcoreMesh(**_MESH),
        scratch_types=[
            pltpu.VMEM((CPW, CHUNK), jnp.int32),
            pltpu.VMEM((CHUNK, D_HID), jnp.float32),
            pltpu.VMEM_SHARED((N_PAD, D_HID), jnp.float32),
        ],
        compiler_params=_SC_PARAMS,
    )
    def deg_kernel(dst_hbm, out_hbm, di_v, ones_v, acc_sh):
        c = lax.axis_index("c")
        s = lax.axis_index("s")
        w = c * NS + s
        row0 = s * RPS
        pltpu.sync_copy(dst_hbm.at[pl.ds(w * CPW, CPW)], di_v)

        @pl.loop(0, CHUNK)
        def _(i):
            ones_v[i] = jnp.zeros((D_HID,), jnp.float32)

        for k in range(RPS // CHUNK):
            pltpu.sync_copy(ones_v, acc_sh.at[pl.ds(row0 + k * CHUNK, CHUNK)])

        @pl.loop(0, CHUNK)
        def _(i):
            ones_v[i] = jnp.ones((D_HID,), jnp.float32)

        plsc.subcore_barrier()

        @pl.loop(0, CPW)
        def _(i):
            pltpu.sync_copy(ones_v, acc_sh.at[di_v.at[i]], add=True)

        plsc.subcore_barrier()
        pltpu.sync_copy(acc_sh.at[pl.ds(row0, RPS)],
                        out_hbm.at[c, pl.ds(row0, RPS)])

    return deg_kernel(dst2d)


def _sc_scatter(table, src2d, dst2d, d, n0, n1):
    @functools.partial(
        pl.kernel,
        out_type=jax.ShapeDtypeStruct((NC, N_PAD, d), jnp.bfloat16),
        mesh=plsc.VectorSubcoreMesh(**_MESH),
        scratch_types=[
            pltpu.VMEM((n0, CHUNK), jnp.int32),
            pltpu.VMEM((n0, CHUNK), jnp.int32),
            pltpu.VMEM((CHUNK, d), jnp.bfloat16),
            pltpu.VMEM((CHUNK, d), jnp.bfloat16),
            pltpu.VMEM_SHARED((N_PAD, d), jnp.bfloat16),
            pltpu.SemaphoreType.DMA,
            pltpu.SemaphoreType.DMA,
        ],
        compiler_params=_SC_PARAMS,
    )
    def scat_kernel(tab_hbm, src_hbm, dst_hbm, out_hbm,
                    si_v, di_v, buf_a, buf_b, acc_sh, sem_a, sem_b):
        c = lax.axis_index("c")
        s = lax.axis_index("s")
        row0 = s * RPS
        my_cpw = lax.select(c == 0, n0, n1)
        base = lax.select(c == 0, s * n0, NS * n0 + s * n1)
        pltpu.sync_copy(src_hbm.at[pl.ds(base, n0)], si_v)
        pltpu.sync_copy(dst_hbm.at[pl.ds(base, n0)], di_v)

        @pl.loop(0, CHUNK // 2)
        def _(i):
            for col in range(d // D_HID):
                buf_a[pl.ds(2 * i, 2), pl.ds(col * D_HID, D_HID)] = (
                    jnp.zeros((2, D_HID), jnp.bfloat16))

        for k in range(RPS // CHUNK):
            pltpu.sync_copy(buf_a, acc_sh.at[pl.ds(row0 + k * CHUNK, CHUNK)])
        plsc.subcore_barrier()

        def gather(i, buf, sem):
            pltpu.async_copy(tab_hbm.at[si_v.at[i]], buf, sem)

        def drain_scatter(i, buf, sem):
            pltpu.make_async_copy(tab_hbm.at[si_v.at[i]], buf, sem).wait()
            pltpu.sync_copy(buf, acc_sh.at[di_v.at[i]], add=True)

        gather(0, buf_a, sem_a)

        @pl.loop(0, my_cpw, step=2)
        def _(i):
            gather(i + 1, buf_b, sem_b)
            drain_scatter(i, buf_a, sem_a)

            @pl.when(i + 2 < my_cpw)
            def _():
                gather(i + 2, buf_a, sem_a)

            drain_scatter(i + 1, buf_b, sem_b)

        plsc.subcore_barrier()
        pltpu.sync_copy(acc_sh.at[pl.ds(row0, RPS)],
                        out_hbm.at[c, pl.ds(row0, RPS)])

    return scat_kernel(table, src2d, dst2d)


_NB = 1024
_V16 = N_PAD * D_HID // 128
_V64 = N_PAD * D_OUT // 128
_B16 = _NB * D_HID // 128
_B64 = _NB * D_OUT // 128


def _tc_first(x_v, w1k, deg_v):
    def body(x_ref, w_ref, dp_ref, u_ref, hn_ref):
        h1v = lax.dot_general(x_ref[...], w_ref[...], _DOT,
                              preferred_element_type=jnp.float32)
        u = lax.rsqrt(dp_ref[0] + dp_ref[1] + 1.0)
        u_ref[...] = u
        hn_ref[...] = (u * h1v).astype(jnp.bfloat16)

    return pl.pallas_call(
        body,
        grid=(N_PAD // _NB,),
        in_specs=[pl.BlockSpec((_B16, 8 * D_IN), lambda i: (i, 0)),
                  pl.BlockSpec((8 * D_IN, 128), lambda i: (0, 0)),
                  pl.BlockSpec((NC, _B16, 128), lambda i: (0, i, 0))],
        out_specs=[pl.BlockSpec((_B16, 128), lambda i: (i, 0)),
                   pl.BlockSpec((_B16, 128), lambda i: (i, 0))],
        out_shape=[jax.ShapeDtypeStruct((_V16, 128), jnp.float32),
                   jax.ShapeDtypeStruct((_V16, 128), jnp.bfloat16)],
    )(x_v, w1k, deg_v)


def _tc_mid(p1v, hn1v, u16v, b1t, w2k, selk):
    def body(p_ref, hn_ref, u_ref, b_ref, w_ref, sel_ref, hn2_ref, u64_ref):
        s1 = (p_ref[0] + p_ref[1]).astype(jnp.float32) + hn_ref[...].astype(jnp.float32)
        pre = u_ref[...] * s1 + b_ref[...]
        o1v = jnp.maximum(pre, 0.0)
        h2w = lax.dot_general(o1v, w_ref[...], _DOT,
                              preferred_element_type=jnp.float32)
        u64w = lax.dot_general(u_ref[...], sel_ref[...], _DOT, precision=_HIGH,
                               preferred_element_type=jnp.float32)
        u64_ref[...] = jnp.reshape(u64w, (_B64, 128))
        hn2_ref[...] = jnp.reshape((u64w * h2w).astype(jnp.bfloat16), (_B64, 128))

    return pl.pallas_call(
        body,
        grid=(N_PAD // _NB,),
        in_specs=[pl.BlockSpec((NC, _B16, 128), lambda i: (0, i, 0)),
                  pl.BlockSpec((_B16, 128), lambda i: (i, 0)),
                  pl.BlockSpec((_B16, 128), lambda i: (i, 0)),
                  pl.BlockSpec((1, 128), lambda i: (0, 0)),
                  pl.BlockSpec((128, 8 * D_OUT), lambda i: (0, 0)),
                  pl.BlockSpec((128, 8 * D_OUT), lambda i: (0, 0))],
        out_specs=[pl.BlockSpec((_B64, 128), lambda i: (i, 0)),
                   pl.BlockSpec((_B64, 128), lambda i: (i, 0))],
        out_shape=[jax.ShapeDtypeStruct((_V64, 128), jnp.bfloat16),
                   jax.ShapeDtypeStruct((_V64, 128), jnp.float32)],
    )(p1v, hn1v, u16v, b1t, w2k, selk)


_FV = N_NODES * D_OUT // 128
_FB = 1000


def _tc_final(p2v, hn2v, u64v, b2t):
    def body(p_ref, hn_ref, u_ref, b_ref, o_ref):
        s2 = (p_ref[0] + p_ref[1]).astype(jnp.float32) + hn_ref[...].astype(jnp.float32)
        o_ref[...] = u_ref[...] * s2 + b_ref[...]

    return pl.pallas_call(
        body,
        grid=(_FV // _FB,),
        in_specs=[pl.BlockSpec((NC, _FB, 128), lambda i: (0, i, 0)),
                  pl.BlockSpec((_FB, 128), lambda i: (i, 0)),
                  pl.BlockSpec((_FB, 128), lambda i: (i, 0)),
                  pl.BlockSpec((1, 128), lambda i: (0, 0))],
        out_specs=pl.BlockSpec((_FB, 128), lambda i: (i, 0)),
        out_shape=jax.ShapeDtypeStruct((_FV, 128), jnp.float32),
    )(p2v, hn2v, u64v, b2t)


def kernel(x, edge_index, W1, b1, W2, b2):
    src = edge_index[0].astype(jnp.int32)
    dst = edge_index[1].astype(jnp.int32)
    pad = E_PAD - N_EDGES
    src2d = jnp.concatenate([src, jnp.zeros((pad,), jnp.int32)]).reshape(N_CHUNKROWS, CHUNK)
    dst2d = jnp.concatenate([dst, jnp.full((pad,), N_NODES, jnp.int32)]).reshape(N_CHUNKROWS, CHUNK)
    x_v = jnp.concatenate(
        [x, jnp.zeros((N_PAD - N_NODES, D_IN), jnp.float32)]).reshape(_V16, 8 * D_IN)
    eye8 = jnp.eye(8, dtype=jnp.float32)
    w1k = jnp.kron(eye8, W1)
    w2k = jnp.kron(eye8, W2)
    selk = jnp.kron(eye8, jnp.zeros((D_HID, D_OUT), jnp.float32).at[0].set(1.0))
    b1t = jnp.tile(b1, 128 // D_HID).reshape(1, 128)
    b2t = jnp.tile(b2, 128 // D_OUT).reshape(1, 128)

    deg_p = _sc_degree(dst2d)
    deg_v = jnp.reshape(deg_p, (NC, _V16, 128))
    u16v, hn1v = _tc_first(x_v, w1k, deg_v)
    p1 = _sc_scatter(hn1v.reshape(N_PAD, D_HID), src2d, dst2d,
                     D_HID, CPW0_L1, CPW1_L1)
    p1v = jnp.reshape(p1, (NC, _V16, 128))
    hn2v, u64v = _tc_mid(p1v, hn1v, u16v, b1t, w2k, selk)
    p2 = _sc_scatter(hn2v.reshape(N_PAD, D_OUT), src2d, dst2d,
                     D_OUT, CPW0_L2, CPW1_L2)
    p2v = jnp.reshape(p2, (NC, _V64, 128))
    out_v = _tc_final(p2v, hn2v, u64v, b2t)
    return jnp.reshape(out_v, (N_NODES, D_OUT))
